# baseline (speedup 1.0000x reference)
import jax
import jax.numpy as jnp
from jax.experimental import pallas as pl
from jax.experimental.pallas import tpu as pltpu

_B, _N, _D, _DE = 8, 512, 768, 16
_NEG = float("-inf")


def _body(asp_ref,
          feat_ref,
          feat4_ref,
          dta_ref,
          dep_ref,
          depT_ref,
          wzT_ref,
          bz_ref,
          wae_ref,
          waf_ref,
          war_ref,
          wmeT_ref,
          wmfT_ref,
          wh1T_ref,
          wh2T_ref,
          out_ref,
          e1_s,
          e2_s,
          d1_s,
          d2_s,
          asp_s,
          af_s,
          sems):
    b = pl.program_id(0)
    i0 = asp_ref[b]
    ia = pl.multiple_of(jnp.minimum((i0 // 8) * 8, _N - 16), 8)
    off = i0 - ia

    cp_out = pltpu.make_async_copy(
        feat4_ref.at[b], out_ref.at[b], sems.at[0])
    cp_out.start()
    cp_e1 = pltpu.make_async_copy(
        dta_ref.at[b, pl.ds(i0, 4), :, :], e1_s, sems.at[1])
    cp_e1.start()
    cp_d1 = pltpu.make_async_copy(
        dep_ref.at[b, pl.ds(i0, 4), :, :], d1_s, sems.at[2])
    cp_d1.start()
    cp_d2 = pltpu.make_async_copy(
        depT_ref.at[b, pl.ds(i0, 4), :, :], d2_s, sems.at[3])
    cp_d2.start()
    cp_e2 = pltpu.make_async_copy(
        dta_ref.at[b, :, pl.ds(ia, 16), :], e2_s, sems.at[4])
    cp_e2.start()
    cp_asp = pltpu.make_async_copy(
        feat4_ref.at[0, pl.ds(i0 + 1, 4), :, :], asp_s, sems.at[5])
    cp_asp.start()

    f_in = feat_ref[0, 1:, :]
    fz = jnp.dot(f_in, wzT_ref[...],
                 preferred_element_type=jnp.float32) + bz_ref[...]
    fs = jnp.dot(fz, waf_ref[...],
                 preferred_element_type=jnp.float32)
    fm = jnp.dot(fz, wmfT_ref[...],
                 preferred_element_type=jnp.float32)

    cp_asp.wait()
    asp = jnp.dot(asp_s[:, 0, :], wzT_ref[...],
                  preferred_element_type=jnp.float32) + bz_ref[...]
    asp_r = jnp.dot(asp, war_ref[...],
                    preferred_element_type=jnp.float32)

    cp_e1.wait()
    cp_d1.wait()
    cp_d2.wait()
    cp_e2.wait()

    col = jax.lax.broadcasted_iota(jnp.int32, (1, 16, 1), 1)
    fused_rows = []
    for k in range(4):
        e1k = e1_s[k]
        onehot = (col == off + k).astype(jnp.float32)
        e2k = jnp.sum(e2_s[...] * onehot, axis=1)
        m1k = d1_s[k] != 0
        m2k = d2_s[k] != 0
        ck = asp_r[k:k + 1, 0:1]
        s1 = jnp.dot(e1k, wae_ref[...],
                     preferred_element_type=jnp.float32) + fs + ck
        s2 = jnp.dot(e2k, wae_ref[...],
                     preferred_element_type=jnp.float32) + fs + ck
        s1 = jnp.where(s1 >= 0, s1, 0.01 * s1)
        s2 = jnp.where(s2 >= 0, s2, 0.01 * s2)
        smax = jnp.maximum(
            jnp.max(jnp.where(m1k, s1, _NEG), axis=0, keepdims=True),
            jnp.max(jnp.where(m2k, s2, _NEG), axis=0, keepdims=True))
        u1 = jnp.where(m1k, jnp.exp(s1 - smax), 0.0)
        u2 = jnp.where(m2k, jnp.exp(s2 - smax), 0.0)
        den = (jnp.sum(u1, axis=0, keepdims=True)
               + jnp.sum(u2, axis=0, keepdims=True))
        den = jnp.where(den == 0.0, 1.0, den)
        w1 = u1 / den
        w2 = u2 / den
        it1 = jax.nn.relu(jnp.dot(e1k, wmeT_ref[...],
                                  preferred_element_type=jnp.float32) + fm)
        it2 = jax.nn.relu(jnp.dot(e2k, wmeT_ref[...],
                                  preferred_element_type=jnp.float32) + fm)
        dn = (((0,), (0,)), ((), ()))
        fused_k = (jax.lax.dot_general(w1, it1, dn,
                                       preferred_element_type=jnp.float32)
                   + jax.lax.dot_general(w2, it2, dn,
                                         preferred_element_type=jnp.float32))
        fused_rows.append(fused_k)

    fused = jnp.concatenate(fused_rows, axis=0)
    af = jax.nn.relu(
        jnp.dot(fused, wh1T_ref[...], preferred_element_type=jnp.float32)
        + jnp.dot(asp, wh2T_ref[...], preferred_element_type=jnp.float32))
    af_s[:, 0, :] = af

    cp_out.wait()
    cp_af = pltpu.make_async_copy(
        af_s, out_ref.at[b, pl.ds(i0 + 1, 4), :, :], sems.at[6])
    cp_af.start()
    cp_af.wait()


def kernel(features, dep_type_adj, text_bert_indices, bert_segments_ids,
           attention_mask, deprel_adj, asp_start, asp_end, src_mask,
           aspect_mask, Wz, bz, Wa, Wm, Wh):
    del text_bert_indices, bert_segments_ids, attention_mask, asp_end
    del src_mask, aspect_mask
    B, N, D, DE = _B, _N, _D, _DE
    asp_i = asp_start.astype(jnp.int32)
    wzT = Wz.T
    bz2 = bz.reshape(1, D)
    wae = Wa[0, :DE].reshape(DE, 1)
    waf = Wa[0, DE:DE + D].reshape(D, 1)
    war = Wa[0, DE + D:].reshape(D, 1)
    wmeT = Wm[:, :DE].T
    wmfT = Wm[:, DE:].T
    wh1T = Wh[:, :D].T
    wh2T = Wh[:, D:].T
    feat4 = features.reshape(B, N + 1, 1, D)
    dep_i = deprel_adj.astype(jnp.int32).reshape(B, N, N, 1)
    dep_t = deprel_adj.astype(jnp.int32).transpose(0, 2, 1).reshape(B, N, N, 1)

    grid_spec = pltpu.PrefetchScalarGridSpec(
        num_scalar_prefetch=1,
        grid=(B,),
        in_specs=[
            pl.BlockSpec((1, N + 1, D), lambda b, a: (b, 0, 0)),
            pl.BlockSpec(memory_space=pl.ANY),
            pl.BlockSpec(memory_space=pl.ANY),
            pl.BlockSpec(memory_space=pl.ANY),
            pl.BlockSpec(memory_space=pl.ANY),
            pl.BlockSpec((D, D), lambda b, a: (0, 0)),
            pl.BlockSpec((1, D), lambda b, a: (0, 0)),
            pl.BlockSpec((DE, 1), lambda b, a: (0, 0)),
            pl.BlockSpec((D, 1), lambda b, a: (0, 0)),
            pl.BlockSpec((D, 1), lambda b, a: (0, 0)),
            pl.BlockSpec((DE, D), lambda b, a: (0, 0)),
            pl.BlockSpec((D, D), lambda b, a: (0, 0)),
            pl.BlockSpec((D, D), lambda b, a: (0, 0)),
            pl.BlockSpec((D, D), lambda b, a: (0, 0)),
        ],
        out_specs=pl.BlockSpec(memory_space=pl.ANY),
        scratch_shapes=[
            pltpu.VMEM((4, N, DE), jnp.float32),
            pltpu.VMEM((N, 16, DE), jnp.float32),
            pltpu.VMEM((4, N, 1), jnp.int32),
            pltpu.VMEM((4, N, 1), jnp.int32),
            pltpu.VMEM((4, 1, D), jnp.float32),
            pltpu.VMEM((4, 1, D), jnp.float32),
            pltpu.SemaphoreType.DMA((7,)),
        ],
    )
    out = pl.pallas_call(
        _body,
        grid_spec=grid_spec,
        out_shape=jax.ShapeDtypeStruct((B, N + 1, 1, D), jnp.float32),
    )(asp_i, features, feat4, dep_type_adj, dep_i, dep_t,
      wzT, bz2, wae, waf, war, wmeT, wmfT, wh1T, wh2T)
    return out.reshape(B, N + 1, D)


# speedup vs baseline: 1.3353x; 1.3353x over previous
import jax
import jax.numpy as jnp
from jax.experimental import pallas as pl
from jax.experimental.pallas import tpu as pltpu

_B, _N, _D, _DE = 8, 512, 768, 16
_NEG = float("-inf")


def _body(asp_ref,
          feat_ref,
          feat4_ref,
          dta_ref,
          dep_ref,
          wzT_ref,
          bz_ref,
          wae_ref,
          waf_ref,
          war_ref,
          wmeT_ref,
          wmfT_ref,
          wh1T_ref,
          wh2T_ref,
          out_ref,
          e1_s,
          e2_s,
          d1_s,
          d2_s,
          asp_s,
          af_s,
          sems):
    b = pl.program_id(0)
    i0 = asp_ref[b]
    ia = pl.multiple_of(jnp.minimum((i0 // 8) * 8, _N - 16), 8)
    off = i0 - ia

    cp_out = pltpu.make_async_copy(
        feat4_ref.at[b], out_ref.at[b], sems.at[0])
    cp_out.start()
    cp_e1 = pltpu.make_async_copy(
        dta_ref.at[b, pl.ds(i0, 4), :, :], e1_s, sems.at[1])
    cp_e1.start()
    cp_d1 = pltpu.make_async_copy(
        dep_ref.at[b, pl.ds(i0, 4), :, :], d1_s, sems.at[2])
    cp_d1.start()
    cp_d2 = pltpu.make_async_copy(
        dep_ref.at[b, :, pl.ds(ia, 16), :], d2_s, sems.at[3])
    cp_d2.start()
    cp_e2 = pltpu.make_async_copy(
        dta_ref.at[b, :, pl.ds(ia, 16), :], e2_s, sems.at[4])
    cp_e2.start()
    cp_asp = pltpu.make_async_copy(
        feat4_ref.at[0, pl.ds(i0 + 1, 4), :, :], asp_s, sems.at[5])
    cp_asp.start()

    f_in = feat_ref[0, 1:, :]
    fz = jnp.dot(f_in, wzT_ref[...],
                 preferred_element_type=jnp.float32) + bz_ref[...]
    fs = jnp.dot(fz, waf_ref[...],
                 preferred_element_type=jnp.float32)
    fm = jnp.dot(fz, wmfT_ref[...],
                 preferred_element_type=jnp.float32)

    cp_asp.wait()
    asp = jnp.dot(asp_s[:, 0, :], wzT_ref[...],
                  preferred_element_type=jnp.float32) + bz_ref[...]
    asp_r = jnp.dot(asp, war_ref[...],
                    preferred_element_type=jnp.float32)

    cp_e1.wait()
    cp_d1.wait()
    cp_d2.wait()
    cp_e2.wait()

    col = jax.lax.broadcasted_iota(jnp.int32, (1, 16, 1), 1)
    fused_rows = []
    for k in range(4):
        e1k = e1_s[k]
        onehot = (col == off + k).astype(jnp.float32)
        e2k = jnp.sum(e2_s[...] * onehot, axis=1)
        m1k = d1_s[k] != 0
        oh_i = (col == off + k).astype(jnp.int32)
        m2k = jnp.sum(d2_s[...] * oh_i, axis=1) != 0
        ck = asp_r[k:k + 1, 0:1]
        s1 = jnp.dot(e1k, wae_ref[...],
                     preferred_element_type=jnp.float32) + fs + ck
        s2 = jnp.dot(e2k, wae_ref[...],
                     preferred_element_type=jnp.float32) + fs + ck
        s1 = jnp.where(s1 >= 0, s1, 0.01 * s1)
        s2 = jnp.where(s2 >= 0, s2, 0.01 * s2)
        smax = jnp.maximum(
            jnp.max(jnp.where(m1k, s1, _NEG), axis=0, keepdims=True),
            jnp.max(jnp.where(m2k, s2, _NEG), axis=0, keepdims=True))
        u1 = jnp.where(m1k, jnp.exp(s1 - smax), 0.0)
        u2 = jnp.where(m2k, jnp.exp(s2 - smax), 0.0)
        den = (jnp.sum(u1, axis=0, keepdims=True)
               + jnp.sum(u2, axis=0, keepdims=True))
        den = jnp.where(den == 0.0, 1.0, den)
        w1 = u1 / den
        w2 = u2 / den
        it1 = jax.nn.relu(jnp.dot(e1k, wmeT_ref[...],
                                  preferred_element_type=jnp.float32) + fm)
        it2 = jax.nn.relu(jnp.dot(e2k, wmeT_ref[...],
                                  preferred_element_type=jnp.float32) + fm)
        dn = (((0,), (0,)), ((), ()))
        fused_k = (jax.lax.dot_general(w1, it1, dn,
                                       preferred_element_type=jnp.float32)
                   + jax.lax.dot_general(w2, it2, dn,
                                         preferred_element_type=jnp.float32))
        fused_rows.append(fused_k)

    fused = jnp.concatenate(fused_rows, axis=0)
    af = jax.nn.relu(
        jnp.dot(fused, wh1T_ref[...], preferred_element_type=jnp.float32)
        + jnp.dot(asp, wh2T_ref[...], preferred_element_type=jnp.float32))
    af_s[:, 0, :] = af

    cp_out.wait()
    cp_af = pltpu.make_async_copy(
        af_s, out_ref.at[b, pl.ds(i0 + 1, 4), :, :], sems.at[6])
    cp_af.start()
    cp_af.wait()


def kernel(features, dep_type_adj, text_bert_indices, bert_segments_ids,
           attention_mask, deprel_adj, asp_start, asp_end, src_mask,
           aspect_mask, Wz, bz, Wa, Wm, Wh):
    del text_bert_indices, bert_segments_ids, attention_mask, asp_end
    del src_mask, aspect_mask
    B, N, D, DE = _B, _N, _D, _DE
    asp_i = asp_start.astype(jnp.int32)
    wzT = Wz.T
    bz2 = bz.reshape(1, D)
    wae = Wa[0, :DE].reshape(DE, 1)
    waf = Wa[0, DE:DE + D].reshape(D, 1)
    war = Wa[0, DE + D:].reshape(D, 1)
    wmeT = Wm[:, :DE].T
    wmfT = Wm[:, DE:].T
    wh1T = Wh[:, :D].T
    wh2T = Wh[:, D:].T
    feat4 = features.reshape(B, N + 1, 1, D)
    dep_i = deprel_adj.astype(jnp.int32).reshape(B, N, N, 1)

    grid_spec = pltpu.PrefetchScalarGridSpec(
        num_scalar_prefetch=1,
        grid=(B,),
        in_specs=[
            pl.BlockSpec((1, N + 1, D), lambda b, a: (b, 0, 0)),
            pl.BlockSpec(memory_space=pl.ANY),
            pl.BlockSpec(memory_space=pl.ANY),
            pl.BlockSpec(memory_space=pl.ANY),
            pl.BlockSpec((D, D), lambda b, a: (0, 0)),
            pl.BlockSpec((1, D), lambda b, a: (0, 0)),
            pl.BlockSpec((DE, 1), lambda b, a: (0, 0)),
            pl.BlockSpec((D, 1), lambda b, a: (0, 0)),
            pl.BlockSpec((D, 1), lambda b, a: (0, 0)),
            pl.BlockSpec((DE, D), lambda b, a: (0, 0)),
            pl.BlockSpec((D, D), lambda b, a: (0, 0)),
            pl.BlockSpec((D, D), lambda b, a: (0, 0)),
            pl.BlockSpec((D, D), lambda b, a: (0, 0)),
        ],
        out_specs=pl.BlockSpec(memory_space=pl.ANY),
        scratch_shapes=[
            pltpu.VMEM((4, N, DE), jnp.float32),
            pltpu.VMEM((N, 16, DE), jnp.float32),
            pltpu.VMEM((4, N, 1), jnp.int32),
            pltpu.VMEM((N, 16, 1), jnp.int32),
            pltpu.VMEM((4, 1, D), jnp.float32),
            pltpu.VMEM((4, 1, D), jnp.float32),
            pltpu.SemaphoreType.DMA((7,)),
        ],
    )
    out = pl.pallas_call(
        _body,
        grid_spec=grid_spec,
        out_shape=jax.ShapeDtypeStruct((B, N + 1, 1, D), jnp.float32),
    )(asp_i, features, feat4, dep_type_adj, dep_i,
      wzT, bz2, wae, waf, war, wmeT, wmfT, wh1T, wh2T)
    return out.reshape(B, N + 1, D)


# speedup vs baseline: 3.4102x; 2.5538x over previous
import jax
import jax.numpy as jnp
from jax.experimental import pallas as pl
from jax.experimental.pallas import tpu as pltpu

_B, _N, _D, _DE = 8, 512, 768, 16
_NEG = float("-inf")


def _body(asp_ref,
          feat_ref,
          feat4_ref,
          dtaT_ref,
          e2c_ref,
          m2c_ref,
          dep_ref,
          wzT_ref,
          bz_ref,
          wae_ref,
          waf_ref,
          war_ref,
          wmeT_ref,
          wmfT_ref,
          wh1T_ref,
          wh2T_ref,
          out_ref,
          fm_s,
          fs_s,
          aspz_s,
          aspr_s,
          asp_s,
          e1t_s,
          m1_s,
          af_s,
          sems):
    b = pl.program_id(0)
    k = pl.program_id(1)
    i0 = asp_ref[b]
    i = i0 + k
    ra = pl.multiple_of(jnp.minimum((i // 8) * 8, _N - 16), 8)

    cp_e1 = pltpu.make_async_copy(dtaT_ref.at[b, i, :, :], e1t_s, sems.at[0])
    cp_e1.start()
    cp_m1 = pltpu.make_async_copy(
        dep_ref.at[b, pl.ds(ra, 16), :], m1_s, sems.at[1])
    cp_m1.start()

    @pl.when(k == 0)
    def _dense():
        cp_out = pltpu.make_async_copy(
            feat4_ref.at[b], out_ref.at[b], sems.at[2])
        cp_out.start()
        cp_asp = pltpu.make_async_copy(
            feat4_ref.at[0, pl.ds(i0 + 1, 4), :, :], asp_s, sems.at[3])
        cp_asp.start()
        f_in = feat_ref[0, 1:, :]
        fz = jnp.dot(f_in, wzT_ref[...],
                     preferred_element_type=jnp.float32) + bz_ref[...]
        fs_s[...] = jnp.dot(fz, waf_ref[...],
                            preferred_element_type=jnp.float32)
        fm_s[...] = jnp.dot(fz, wmfT_ref[...],
                            preferred_element_type=jnp.float32)
        cp_asp.wait()
        aspz = jnp.dot(asp_s[:, 0, :], wzT_ref[...],
                       preferred_element_type=jnp.float32) + bz_ref[...]
        aspz_s[...] = aspz
        aspr_s[...] = jnp.dot(aspz, war_ref[...],
                              preferred_element_type=jnp.float32)
        cp_out.wait()

    dn0 = (((0,), (0,)), ((), ()))
    lane = jax.lax.rem(i, 128)
    l_iota = jax.lax.broadcasted_iota(jnp.int32, (1, 1, 128), 2)
    e2k = jnp.sum(e2c_ref[0] * (l_iota == lane).astype(jnp.float32),
                  axis=2)
    m_iota = jax.lax.broadcasted_iota(jnp.int32, (1, 128), 1)
    m2k = (jnp.sum(m2c_ref[0] * (m_iota == lane).astype(jnp.int32),
                   axis=1, keepdims=True) != 0)

    cp_e1.wait()
    cp_m1.wait()
    s_iota = jax.lax.broadcasted_iota(jnp.int32, (16, 1), 0)
    m1row = jnp.sum(m1_s[...] * (s_iota == i - ra).astype(jnp.int32),
                    axis=0, keepdims=True)
    ones11 = jnp.full((1, 1), 1.0, jnp.float32)
    m1colf = jax.lax.dot_general((m1row != 0).astype(jnp.float32), ones11,
                                 dn0, preferred_element_type=jnp.float32)
    m1k = m1colf > 0.5

    e1t = e1t_s[...]
    fs = fs_s[...]
    fm = fm_s[...]
    k_iota = jax.lax.broadcasted_iota(jnp.int32, (4, 1), 0)
    k_hot = (k_iota == k).astype(jnp.float32)
    ck = jnp.sum(aspr_s[...] * k_hot, axis=0, keepdims=True)

    s1 = jax.lax.dot_general(e1t, wae_ref[...], dn0,
                             preferred_element_type=jnp.float32) + fs + ck
    s2 = jnp.dot(e2k, wae_ref[...],
                 preferred_element_type=jnp.float32) + fs + ck
    s1 = jnp.where(s1 >= 0, s1, 0.01 * s1)
    s2 = jnp.where(s2 >= 0, s2, 0.01 * s2)
    smax = jnp.maximum(
        jnp.max(jnp.where(m1k, s1, _NEG), axis=0, keepdims=True),
        jnp.max(jnp.where(m2k, s2, _NEG), axis=0, keepdims=True))
    u1 = jnp.where(m1k, jnp.exp(s1 - smax), 0.0)
    u2 = jnp.where(m2k, jnp.exp(s2 - smax), 0.0)
    den = (jnp.sum(u1, axis=0, keepdims=True)
           + jnp.sum(u2, axis=0, keepdims=True))
    den = jnp.where(den == 0.0, 1.0, den)
    w1 = u1 / den
    w2 = u2 / den

    it1 = jax.nn.relu(jax.lax.dot_general(
        e1t, wmeT_ref[...], dn0, preferred_element_type=jnp.float32) + fm)
    it2 = jax.nn.relu(jnp.dot(e2k, wmeT_ref[...],
                              preferred_element_type=jnp.float32) + fm)
    fused = jnp.sum(w1 * it1 + w2 * it2, axis=0, keepdims=True)

    aspz_k = jnp.sum(aspz_s[...] * k_hot, axis=0, keepdims=True)
    af = jax.nn.relu(
        jnp.dot(fused, wh1T_ref[...], preferred_element_type=jnp.float32)
        + jnp.dot(aspz_k, wh2T_ref[...],
                  preferred_element_type=jnp.float32))
    af_s[0] = af
    cp_af = pltpu.make_async_copy(
        af_s, out_ref.at[b, pl.ds(i + 1, 1), :, :], sems.at[4])
    cp_af.start()
    cp_af.wait()


def kernel(features, dep_type_adj, text_bert_indices, bert_segments_ids,
           attention_mask, deprel_adj, asp_start, asp_end, src_mask,
           aspect_mask, Wz, bz, Wa, Wm, Wh):
    del text_bert_indices, bert_segments_ids, attention_mask, asp_end
    del src_mask, aspect_mask
    B, N, D, DE = _B, _N, _D, _DE
    asp_i = asp_start.astype(jnp.int32)
    wzT = Wz.T
    bz2 = bz.reshape(1, D)
    wae = Wa[0, :DE].reshape(DE, 1)
    waf = Wa[0, DE:DE + D].reshape(D, 1)
    war = Wa[0, DE + D:].reshape(D, 1)
    wmeT = Wm[:, :DE].T
    wmfT = Wm[:, DE:].T
    wh1T = Wh[:, :D].T
    wh2T = Wh[:, D:].T
    feat4 = features.reshape(B, N + 1, 1, D)
    dtaT = dep_type_adj.transpose(0, 1, 3, 2)
    dep_i = deprel_adj.astype(jnp.int32)

    grid_spec = pltpu.PrefetchScalarGridSpec(
        num_scalar_prefetch=1,
        grid=(B, 4),
        in_specs=[
            pl.BlockSpec((1, N + 1, D), lambda b, k, a: (b, 0, 0)),
            pl.BlockSpec(memory_space=pl.ANY),
            pl.BlockSpec(memory_space=pl.ANY),
            pl.BlockSpec((1, N, DE, 128),
                         lambda b, k, a: (b, 0, 0, (a[b] + k) // 128)),
            pl.BlockSpec((1, N, 128),
                         lambda b, k, a: (b, 0, (a[b] + k) // 128)),
            pl.BlockSpec(memory_space=pl.ANY),
            pl.BlockSpec((D, D), lambda b, k, a: (0, 0)),
            pl.BlockSpec((1, D), lambda b, k, a: (0, 0)),
            pl.BlockSpec((DE, 1), lambda b, k, a: (0, 0)),
            pl.BlockSpec((D, 1), lambda b, k, a: (0, 0)),
            pl.BlockSpec((D, 1), lambda b, k, a: (0, 0)),
            pl.BlockSpec((DE, D), lambda b, k, a: (0, 0)),
            pl.BlockSpec((D, D), lambda b, k, a: (0, 0)),
            pl.BlockSpec((D, D), lambda b, k, a: (0, 0)),
            pl.BlockSpec((D, D), lambda b, k, a: (0, 0)),
        ],
        out_specs=pl.BlockSpec(memory_space=pl.ANY),
        scratch_shapes=[
            pltpu.VMEM((N, D), jnp.float32),
            pltpu.VMEM((N, 1), jnp.float32),
            pltpu.VMEM((4, D), jnp.float32),
            pltpu.VMEM((4, 1), jnp.float32),
            pltpu.VMEM((4, 1, D), jnp.float32),
            pltpu.VMEM((DE, N), jnp.float32),
            pltpu.VMEM((16, N), jnp.int32),
            pltpu.VMEM((1, 1, D), jnp.float32),
            pltpu.SemaphoreType.DMA((5,)),
        ],
    )
    out = pl.pallas_call(
        _body,
        grid_spec=grid_spec,
        out_shape=jax.ShapeDtypeStruct((B, N + 1, 1, D), jnp.float32),
    )(asp_i, features, feat4, dtaT, dtaT, dep_i, dep_i,
      wzT, bz2, wae, waf, war, wmeT, wmfT, wh1T, wh2T)
    return out.reshape(B, N + 1, D)


# speedup vs baseline: 3.4102x; 1.0000x over previous
import jax
import jax.numpy as jnp
from jax.experimental import pallas as pl
from jax.experimental.pallas import tpu as pltpu

_B, _N, _D, _DE = 8, 512, 768, 16
_NEG = float("-inf")


def _body(asp_ref,
          feat_ref,
          feat4_ref,
          dtaT_ref,
          e2c_ref,
          m2c_ref,
          dep_ref,
          wzT_ref,
          bz_ref,
          wae_ref,
          waf_ref,
          war_ref,
          wmeT_ref,
          wmfT_ref,
          wh1T_ref,
          wh2T_ref,
          out_ref,
          fm_s,
          fs_s,
          aspz_s,
          aspr_s,
          asp_s,
          e1t_s,
          m1_s,
          af_s,
          sems):
    b = pl.program_id(0)
    k = pl.program_id(1)
    i0 = asp_ref[b]
    i = i0 + k
    ra = pl.multiple_of(jnp.minimum((i // 8) * 8, _N - 16), 8)

    cp_e1 = pltpu.make_async_copy(dtaT_ref.at[b, i, :, :], e1t_s, sems.at[0])
    cp_e1.start()
    cp_m1 = pltpu.make_async_copy(
        dep_ref.at[b, pl.ds(ra, 16), :], m1_s, sems.at[1])
    cp_m1.start()

    @pl.when(k == 0)
    def _dense():
        cp_out = pltpu.make_async_copy(
            feat4_ref.at[b], out_ref.at[b], sems.at[2])
        cp_out.start()
        cp_asp = pltpu.make_async_copy(
            feat4_ref.at[0, pl.ds(i0 + 1, 4), :, :], asp_s, sems.at[3])
        cp_asp.start()
        f_in = feat_ref[0, 1:, :].astype(jnp.bfloat16)
        fz = jnp.dot(f_in, wzT_ref[...],
                     preferred_element_type=jnp.float32) + bz_ref[...]
        fs_s[...] = jnp.dot(fz, waf_ref[...],
                            preferred_element_type=jnp.float32)
        fm_s[...] = jnp.dot(fz.astype(jnp.bfloat16), wmfT_ref[...],
                            preferred_element_type=jnp.float32)
        cp_asp.wait()
        aspz = jnp.dot(asp_s[:, 0, :].astype(jnp.bfloat16), wzT_ref[...],
                       preferred_element_type=jnp.float32) + bz_ref[...]
        aspz_s[...] = aspz
        aspr_s[...] = jnp.dot(aspz, war_ref[...],
                              preferred_element_type=jnp.float32)
        cp_out.wait()

    dn0 = (((0,), (0,)), ((), ()))
    lane = jax.lax.rem(i, 128)
    l_iota = jax.lax.broadcasted_iota(jnp.int32, (1, 1, 128), 2)
    e2k = jnp.sum(e2c_ref[0] * (l_iota == lane).astype(jnp.float32),
                  axis=2)
    m_iota = jax.lax.broadcasted_iota(jnp.int32, (1, 128), 1)
    m2k = (jnp.sum(m2c_ref[0] * (m_iota == lane).astype(jnp.int32),
                   axis=1, keepdims=True) != 0)

    cp_e1.wait()
    cp_m1.wait()
    s_iota = jax.lax.broadcasted_iota(jnp.int32, (16, 1), 0)
    m1row = jnp.sum(m1_s[...] * (s_iota == i - ra).astype(jnp.int32),
                    axis=0, keepdims=True)
    ones11 = jnp.full((1, 1), 1.0, jnp.float32)
    m1colf = jax.lax.dot_general((m1row != 0).astype(jnp.float32), ones11,
                                 dn0, preferred_element_type=jnp.float32)
    m1k = m1colf > 0.5

    e1t = e1t_s[...]
    fs = fs_s[...]
    fm = fm_s[...]
    k_iota = jax.lax.broadcasted_iota(jnp.int32, (4, 1), 0)
    k_hot = (k_iota == k).astype(jnp.float32)
    ck = jnp.sum(aspr_s[...] * k_hot, axis=0, keepdims=True)

    s1 = jax.lax.dot_general(e1t, wae_ref[...], dn0,
                             preferred_element_type=jnp.float32) + fs + ck
    s2 = jnp.dot(e2k, wae_ref[...],
                 preferred_element_type=jnp.float32) + fs + ck
    s1 = jnp.where(s1 >= 0, s1, 0.01 * s1)
    s2 = jnp.where(s2 >= 0, s2, 0.01 * s2)
    smax = jnp.maximum(
        jnp.max(jnp.where(m1k, s1, _NEG), axis=0, keepdims=True),
        jnp.max(jnp.where(m2k, s2, _NEG), axis=0, keepdims=True))
    u1 = jnp.where(m1k, jnp.exp(s1 - smax), 0.0)
    u2 = jnp.where(m2k, jnp.exp(s2 - smax), 0.0)
    den = (jnp.sum(u1, axis=0, keepdims=True)
           + jnp.sum(u2, axis=0, keepdims=True))
    den = jnp.where(den == 0.0, 1.0, den)
    w1 = u1 / den
    w2 = u2 / den

    it1 = jax.nn.relu(jax.lax.dot_general(
        e1t, wmeT_ref[...], dn0, preferred_element_type=jnp.float32) + fm)
    it2 = jax.nn.relu(jnp.dot(e2k, wmeT_ref[...],
                              preferred_element_type=jnp.float32) + fm)
    fused = jnp.sum(w1 * it1 + w2 * it2, axis=0, keepdims=True)

    aspz_k = jnp.sum(aspz_s[...] * k_hot, axis=0, keepdims=True)
    af = jax.nn.relu(
        jnp.dot(fused, wh1T_ref[...], preferred_element_type=jnp.float32)
        + jnp.dot(aspz_k, wh2T_ref[...],
                  preferred_element_type=jnp.float32))
    af_s[0] = af
    cp_af = pltpu.make_async_copy(
        af_s, out_ref.at[b, pl.ds(i + 1, 1), :, :], sems.at[4])
    cp_af.start()
    cp_af.wait()


def kernel(features, dep_type_adj, text_bert_indices, bert_segments_ids,
           attention_mask, deprel_adj, asp_start, asp_end, src_mask,
           aspect_mask, Wz, bz, Wa, Wm, Wh):
    del text_bert_indices, bert_segments_ids, attention_mask, asp_end
    del src_mask, aspect_mask
    B, N, D, DE = _B, _N, _D, _DE
    asp_i = asp_start.astype(jnp.int32)
    wzT = Wz.T.astype(jnp.bfloat16)
    bz2 = bz.reshape(1, D)
    wae = Wa[0, :DE].reshape(DE, 1)
    waf = Wa[0, DE:DE + D].reshape(D, 1)
    war = Wa[0, DE + D:].reshape(D, 1)
    wmeT = Wm[:, :DE].T
    wmfT = Wm[:, DE:].T.astype(jnp.bfloat16)
    wh1T = Wh[:, :D].T
    wh2T = Wh[:, D:].T
    feat4 = features.reshape(B, N + 1, 1, D)
    dtaT = dep_type_adj.transpose(0, 1, 3, 2)
    dep_i = deprel_adj.astype(jnp.int32)

    grid_spec = pltpu.PrefetchScalarGridSpec(
        num_scalar_prefetch=1,
        grid=(B, 4),
        in_specs=[
            pl.BlockSpec((1, N + 1, D), lambda b, k, a: (b, 0, 0)),
            pl.BlockSpec(memory_space=pl.ANY),
            pl.BlockSpec(memory_space=pl.ANY),
            pl.BlockSpec((1, N, DE, 128),
                         lambda b, k, a: (b, 0, 0, (a[b] + k) // 128)),
            pl.BlockSpec((1, N, 128),
                         lambda b, k, a: (b, 0, (a[b] + k) // 128)),
            pl.BlockSpec(memory_space=pl.ANY),
            pl.BlockSpec((D, D), lambda b, k, a: (0, 0)),
            pl.BlockSpec((1, D), lambda b, k, a: (0, 0)),
            pl.BlockSpec((DE, 1), lambda b, k, a: (0, 0)),
            pl.BlockSpec((D, 1), lambda b, k, a: (0, 0)),
            pl.BlockSpec((D, 1), lambda b, k, a: (0, 0)),
            pl.BlockSpec((DE, D), lambda b, k, a: (0, 0)),
            pl.BlockSpec((D, D), lambda b, k, a: (0, 0)),
            pl.BlockSpec((D, D), lambda b, k, a: (0, 0)),
            pl.BlockSpec((D, D), lambda b, k, a: (0, 0)),
        ],
        out_specs=pl.BlockSpec(memory_space=pl.ANY),
        scratch_shapes=[
            pltpu.VMEM((N, D), jnp.float32),
            pltpu.VMEM((N, 1), jnp.float32),
            pltpu.VMEM((4, D), jnp.float32),
            pltpu.VMEM((4, 1), jnp.float32),
            pltpu.VMEM((4, 1, D), jnp.float32),
            pltpu.VMEM((DE, N), jnp.float32),
            pltpu.VMEM((16, N), jnp.int32),
            pltpu.VMEM((1, 1, D), jnp.float32),
            pltpu.SemaphoreType.DMA((5,)),
        ],
    )
    out = pl.pallas_call(
        _body,
        grid_spec=grid_spec,
        out_shape=jax.ShapeDtypeStruct((B, N + 1, 1, D), jnp.float32),
    )(asp_i, features, feat4, dtaT, dtaT, dep_i, dep_i,
      wzT, bz2, wae, waf, war, wmeT, wmfT, wh1T, wh2T)
    return out.reshape(B, N + 1, D)


# speedup vs baseline: 4.3073x; 1.2630x over previous
import jax
import jax.numpy as jnp
from jax.experimental import pallas as pl
from jax.experimental.pallas import tpu as pltpu

_B, _N, _D, _DE = 8, 512, 768, 16
_NEG = float("-inf")


def _body(asp_ref,
          feat_ref,
          feat3_ref,
          dtaT_ref,
          e2c_ref,
          m2c_ref,
          dep_ref,
          wzT_ref,
          bz_ref,
          wae_ref,
          waf_ref,
          war_ref,
          wmeT_ref,
          wmfT_ref,
          wh1T_ref,
          wh2T_ref,
          out_ref,
          fm_s,
          fs_s,
          fz0_s,
          aspz_s,
          aspr_s,
          af4_s,
          e1t_s,
          m1_s,
          rmw_s,
          sems):
    b = pl.program_id(0)
    k = pl.program_id(1)
    i0 = asp_ref[b]
    i = i0 + k
    ra = pl.multiple_of(jnp.minimum((i // 8) * 8, _N - 16), 8)

    cp_e1 = pltpu.make_async_copy(dtaT_ref.at[b, i, :, :], e1t_s, sems.at[0])
    cp_e1.start()
    cp_m1 = pltpu.make_async_copy(
        dep_ref.at[b, pl.ds(ra, 16), :], m1_s, sems.at[1])
    cp_m1.start()

    @pl.when(k == 0)
    def _dense():
        cp_out = pltpu.make_async_copy(
            feat3_ref.at[b], out_ref.at[b], sems.at[2])
        cp_out.start()
        f_in = feat_ref[0, 1:, :].astype(jnp.bfloat16)
        fz = jnp.dot(f_in, wzT_ref[...],
                     preferred_element_type=jnp.float32) + bz_ref[...]
        fs_s[...] = jnp.dot(fz, waf_ref[...],
                            preferred_element_type=jnp.float32)
        fm_s[...] = jnp.dot(fz.astype(jnp.bfloat16), wmfT_ref[...],
                            preferred_element_type=jnp.float32)

        @pl.when(b == 0)
        def _save_fz0():
            fz0_s[...] = fz

        r_iota = jax.lax.broadcasted_iota(jnp.int32, (4, _N), 1)
        k4 = jax.lax.broadcasted_iota(jnp.int32, (4, _N), 0)
        p4 = (r_iota == i0 + k4).astype(jnp.float32)
        aspz = jnp.dot(p4, fz0_s[...],
                       preferred_element_type=jnp.float32)
        aspz_s[...] = aspz
        aspr_s[...] = jnp.dot(aspz, war_ref[...],
                              preferred_element_type=jnp.float32)

    dn0 = (((0,), (0,)), ((), ()))
    lane = jax.lax.rem(i, 128)
    l_iota = jax.lax.broadcasted_iota(jnp.int32, (1, 1, 128), 2)
    e2k = jnp.sum(e2c_ref[0] * (l_iota == lane).astype(jnp.float32),
                  axis=2)
    m_iota = jax.lax.broadcasted_iota(jnp.int32, (1, 128), 1)
    m2k = (jnp.sum(m2c_ref[0] * (m_iota == lane).astype(jnp.int32),
                   axis=1, keepdims=True) != 0)

    cp_e1.wait()
    cp_m1.wait()
    s_iota = jax.lax.broadcasted_iota(jnp.int32, (16, 1), 0)
    m1row = jnp.sum(m1_s[...] * (s_iota == i - ra).astype(jnp.int32),
                    axis=0, keepdims=True)
    ones11 = jnp.full((1, 1), 1.0, jnp.float32)
    m1colf = jax.lax.dot_general((m1row != 0).astype(jnp.float32), ones11,
                                 dn0, preferred_element_type=jnp.float32)
    m1k = m1colf > 0.5

    e1t = e1t_s[...]
    fs = fs_s[...]
    fm = fm_s[...]
    k_iota = jax.lax.broadcasted_iota(jnp.int32, (4, 1), 0)
    k_hot = (k_iota == k).astype(jnp.float32)
    ck = jnp.sum(aspr_s[...] * k_hot, axis=0, keepdims=True)

    s1 = jax.lax.dot_general(e1t, wae_ref[...], dn0,
                             preferred_element_type=jnp.float32) + fs + ck
    s2 = jnp.dot(e2k, wae_ref[...],
                 preferred_element_type=jnp.float32) + fs + ck
    s1 = jnp.where(s1 >= 0, s1, 0.01 * s1)
    s2 = jnp.where(s2 >= 0, s2, 0.01 * s2)
    smax = jnp.maximum(
        jnp.max(jnp.where(m1k, s1, _NEG), axis=0, keepdims=True),
        jnp.max(jnp.where(m2k, s2, _NEG), axis=0, keepdims=True))
    u1 = jnp.where(m1k, jnp.exp(s1 - smax), 0.0)
    u2 = jnp.where(m2k, jnp.exp(s2 - smax), 0.0)
    den = (jnp.sum(u1, axis=0, keepdims=True)
           + jnp.sum(u2, axis=0, keepdims=True))
    den = jnp.where(den == 0.0, 1.0, den)
    w1 = u1 / den
    w2 = u2 / den

    it1 = jax.nn.relu(jax.lax.dot_general(
        e1t, wmeT_ref[...], dn0, preferred_element_type=jnp.float32) + fm)
    it2 = jax.nn.relu(jnp.dot(e2k, wmeT_ref[...],
                              preferred_element_type=jnp.float32) + fm)
    fused = jnp.sum(w1 * it1 + w2 * it2, axis=0, keepdims=True)

    aspz_k = jnp.sum(aspz_s[...] * k_hot, axis=0, keepdims=True)
    af = jax.nn.relu(
        jnp.dot(fused, wh1T_ref[...], preferred_element_type=jnp.float32)
        + jnp.dot(aspz_k, wh2T_ref[...],
                  preferred_element_type=jnp.float32))
    af4_s[...] = jnp.where(k_hot > 0.5, af, af4_s[...])

    @pl.when(k == 3)
    def _writeback():
        wa = pl.multiple_of(jnp.minimum(((i0 + 1) // 8) * 8, _N - 16), 8)
        off = i0 + 1 - wa
        pltpu.make_async_copy(feat3_ref.at[b], out_ref.at[b],
                              sems.at[2]).wait()
        cp_r = pltpu.make_async_copy(
            out_ref.at[b, pl.ds(wa, 16), :], rmw_s, sems.at[3])
        cp_r.start()
        cp_r.wait()
        r16 = jax.lax.broadcasted_iota(jnp.int32, (16, 1), 0)
        c4 = jax.lax.broadcasted_iota(jnp.int32, (1, 4), 1)
        p16 = (r16 - off == c4).astype(jnp.float32)
        af_perm = jnp.dot(p16, af4_s[...],
                          preferred_element_type=jnp.float32)
        sel = (r16 >= off) & (r16 < off + 4)
        rmw_s[...] = jnp.where(sel, af_perm, rmw_s[...])
        cp_w = pltpu.make_async_copy(
            rmw_s, out_ref.at[b, pl.ds(wa, 16), :], sems.at[4])
        cp_w.start()
        cp_w.wait()


def kernel(features, dep_type_adj, text_bert_indices, bert_segments_ids,
           attention_mask, deprel_adj, asp_start, asp_end, src_mask,
           aspect_mask, Wz, bz, Wa, Wm, Wh):
    del text_bert_indices, bert_segments_ids, attention_mask, asp_end
    del src_mask, aspect_mask
    B, N, D, DE = _B, _N, _D, _DE
    asp_i = asp_start.astype(jnp.int32)
    wzT = Wz.T.astype(jnp.bfloat16)
    bz2 = bz.reshape(1, D)
    wae = Wa[0, :DE].reshape(DE, 1)
    waf = Wa[0, DE:DE + D].reshape(D, 1)
    war = Wa[0, DE + D:].reshape(D, 1)
    wmeT = Wm[:, :DE].T
    wmfT = Wm[:, DE:].T.astype(jnp.bfloat16)
    wh1T = Wh[:, :D].T
    wh2T = Wh[:, D:].T
    dtaT = dep_type_adj.transpose(0, 1, 3, 2)
    dep_i = deprel_adj.astype(jnp.int32)

    grid_spec = pltpu.PrefetchScalarGridSpec(
        num_scalar_prefetch=1,
        grid=(B, 4),
        in_specs=[
            pl.BlockSpec((1, N + 1, D), lambda b, k, a: (b, 0, 0)),
            pl.BlockSpec(memory_space=pl.ANY),
            pl.BlockSpec(memory_space=pl.ANY),
            pl.BlockSpec((1, N, DE, 128),
                         lambda b, k, a: (b, 0, 0, (a[b] + k) // 128)),
            pl.BlockSpec((1, N, 128),
                         lambda b, k, a: (b, 0, (a[b] + k) // 128)),
            pl.BlockSpec(memory_space=pl.ANY),
            pl.BlockSpec((D, D), lambda b, k, a: (0, 0)),
            pl.BlockSpec((1, D), lambda b, k, a: (0, 0)),
            pl.BlockSpec((DE, 1), lambda b, k, a: (0, 0)),
            pl.BlockSpec((D, 1), lambda b, k, a: (0, 0)),
            pl.BlockSpec((D, 1), lambda b, k, a: (0, 0)),
            pl.BlockSpec((DE, D), lambda b, k, a: (0, 0)),
            pl.BlockSpec((D, D), lambda b, k, a: (0, 0)),
            pl.BlockSpec((D, D), lambda b, k, a: (0, 0)),
            pl.BlockSpec((D, D), lambda b, k, a: (0, 0)),
        ],
        out_specs=pl.BlockSpec(memory_space=pl.ANY),
        scratch_shapes=[
            pltpu.VMEM((N, D), jnp.float32),
            pltpu.VMEM((N, 1), jnp.float32),
            pltpu.VMEM((N, D), jnp.float32),
            pltpu.VMEM((4, D), jnp.float32),
            pltpu.VMEM((4, 1), jnp.float32),
            pltpu.VMEM((4, D), jnp.float32),
            pltpu.VMEM((DE, N), jnp.float32),
            pltpu.VMEM((16, N), jnp.int32),
            pltpu.VMEM((16, D), jnp.float32),
            pltpu.SemaphoreType.DMA((5,)),
        ],
    )
    out = pl.pallas_call(
        _body,
        grid_spec=grid_spec,
        out_shape=jax.ShapeDtypeStruct((B, N + 1, D), jnp.float32),
    )(asp_i, features, features, dtaT, dtaT, dep_i, dep_i,
      wzT, bz2, wae, waf, war, wmeT, wmfT, wh1T, wh2T)
    return out


# speedup vs baseline: 11.0475x; 2.5648x over previous
import jax
import jax.numpy as jnp
from jax.experimental import pallas as pl
from jax.experimental.pallas import tpu as pltpu

_B, _N, _D, _DE = 8, 512, 768, 16
_NEG = float("-inf")


def _body(asp_ref,
          feat_ref,
          dtaT_ref,
          e2c_ref,
          m2c_ref,
          dep_ref,
          wzT_ref,
          bz_ref,
          wae_ref,
          waf_ref,
          war_ref,
          wmeT_ref,
          wmfT_ref,
          wh1T_ref,
          wh2T_ref,
          out_ref,
          fm_s,
          fs_s,
          fz0_s,
          aspz_s,
          aspr_s,
          af4_s,
          e1t_s,
          m1_s,
          sems):
    b = pl.program_id(0)
    k = pl.program_id(1)
    i0 = asp_ref[b]
    i = i0 + k
    ra = pl.multiple_of(jnp.minimum((i // 8) * 8, _N - 16), 8)

    cp_e1 = pltpu.make_async_copy(dtaT_ref.at[b, i, :, :], e1t_s, sems.at[0])
    cp_e1.start()
    cp_m1 = pltpu.make_async_copy(
        dep_ref.at[b, pl.ds(ra, 16), :], m1_s, sems.at[1])
    cp_m1.start()

    @pl.when(k == 0)
    def _dense():
        f_in = feat_ref[0, 1:, :].astype(jnp.bfloat16)
        fz = jnp.dot(f_in, wzT_ref[...],
                     preferred_element_type=jnp.float32) + bz_ref[...]
        fs_s[...] = jnp.dot(fz, waf_ref[...],
                            preferred_element_type=jnp.float32)
        fm_s[...] = jnp.dot(fz.astype(jnp.bfloat16), wmfT_ref[...],
                            preferred_element_type=jnp.float32)

        @pl.when(b == 0)
        def _save_fz0():
            fz0_s[...] = fz

        r_iota = jax.lax.broadcasted_iota(jnp.int32, (4, _N), 1)
        k4 = jax.lax.broadcasted_iota(jnp.int32, (4, _N), 0)
        p4 = (r_iota == i0 + k4).astype(jnp.float32)
        aspz = jnp.dot(p4, fz0_s[...],
                       preferred_element_type=jnp.float32)
        aspz_s[...] = aspz
        aspr_s[...] = jnp.dot(aspz, war_ref[...],
                              preferred_element_type=jnp.float32)

    dn0 = (((0,), (0,)), ((), ()))
    lane = jax.lax.rem(i, 128)
    l_iota = jax.lax.broadcasted_iota(jnp.int32, (1, 1, 128), 2)
    e2k = jnp.sum(e2c_ref[0] * (l_iota == lane).astype(jnp.float32),
                  axis=2)
    m_iota = jax.lax.broadcasted_iota(jnp.int32, (1, 128), 1)
    m2k = (jnp.sum(m2c_ref[0] * (m_iota == lane).astype(jnp.int32),
                   axis=1, keepdims=True) != 0)

    cp_e1.wait()
    cp_m1.wait()
    s_iota = jax.lax.broadcasted_iota(jnp.int32, (16, 1), 0)
    m1row = jnp.sum(m1_s[...] * (s_iota == i - ra).astype(jnp.int32),
                    axis=0, keepdims=True)
    ones11 = jnp.full((1, 1), 1.0, jnp.float32)
    m1colf = jax.lax.dot_general((m1row != 0).astype(jnp.float32), ones11,
                                 dn0, preferred_element_type=jnp.float32)
    m1k = m1colf > 0.5

    e1t = e1t_s[...]
    fs = fs_s[...]
    fm = fm_s[...]
    k_iota = jax.lax.broadcasted_iota(jnp.int32, (4, 1), 0)
    k_hot = (k_iota == k).astype(jnp.float32)
    ck = jnp.sum(aspr_s[...] * k_hot, axis=0, keepdims=True)

    s1 = jax.lax.dot_general(e1t, wae_ref[...], dn0,
                             preferred_element_type=jnp.float32) + fs + ck
    s2 = jnp.dot(e2k, wae_ref[...],
                 preferred_element_type=jnp.float32) + fs + ck
    s1 = jnp.where(s1 >= 0, s1, 0.01 * s1)
    s2 = jnp.where(s2 >= 0, s2, 0.01 * s2)
    smax = jnp.maximum(
        jnp.max(jnp.where(m1k, s1, _NEG), axis=0, keepdims=True),
        jnp.max(jnp.where(m2k, s2, _NEG), axis=0, keepdims=True))
    u1 = jnp.where(m1k, jnp.exp(s1 - smax), 0.0)
    u2 = jnp.where(m2k, jnp.exp(s2 - smax), 0.0)
    den = (jnp.sum(u1, axis=0, keepdims=True)
           + jnp.sum(u2, axis=0, keepdims=True))
    den = jnp.where(den == 0.0, 1.0, den)
    w1 = u1 / den
    w2 = u2 / den

    it1 = jax.nn.relu(jax.lax.dot_general(
        e1t, wmeT_ref[...], dn0, preferred_element_type=jnp.float32) + fm)
    it2 = jax.nn.relu(jnp.dot(e2k, wmeT_ref[...],
                              preferred_element_type=jnp.float32) + fm)
    fused = jnp.sum(w1 * it1 + w2 * it2, axis=0, keepdims=True)

    aspz_k = jnp.sum(aspz_s[...] * k_hot, axis=0, keepdims=True)
    af = jax.nn.relu(
        jnp.dot(fused, wh1T_ref[...], preferred_element_type=jnp.float32)
        + jnp.dot(aspz_k, wh2T_ref[...],
                  preferred_element_type=jnp.float32))
    af4_s[...] = jnp.where(k_hot > 0.5, af, af4_s[...])

    @pl.when(k == 3)
    def _writeback():
        out_ref[0, :, :] = feat_ref[0, :, :]
        wa = pl.multiple_of(jnp.minimum(((i0 + 1) // 8) * 8, _N - 16), 8)
        off = i0 + 1 - wa
        r16 = jax.lax.broadcasted_iota(jnp.int32, (16, 1), 0)
        c4 = jax.lax.broadcasted_iota(jnp.int32, (1, 4), 1)
        p16 = (r16 - off == c4).astype(jnp.float32)
        af_perm = jnp.dot(p16, af4_s[...],
                          preferred_element_type=jnp.float32)
        sel = (r16 >= off) & (r16 < off + 4)
        win = jnp.where(sel, af_perm, feat_ref[0, pl.ds(wa, 16), :])
        out_ref[0, pl.ds(wa, 16), :] = win


def kernel(features, dep_type_adj, text_bert_indices, bert_segments_ids,
           attention_mask, deprel_adj, asp_start, asp_end, src_mask,
           aspect_mask, Wz, bz, Wa, Wm, Wh):
    del text_bert_indices, bert_segments_ids, attention_mask, asp_end
    del src_mask, aspect_mask
    B, N, D, DE = _B, _N, _D, _DE
    asp_i = asp_start.astype(jnp.int32)
    wzT = Wz.T.astype(jnp.bfloat16)
    bz2 = bz.reshape(1, D)
    wae = Wa[0, :DE].reshape(DE, 1)
    waf = Wa[0, DE:DE + D].reshape(D, 1)
    war = Wa[0, DE + D:].reshape(D, 1)
    wmeT = Wm[:, :DE].T
    wmfT = Wm[:, DE:].T.astype(jnp.bfloat16)
    wh1T = Wh[:, :D].T
    wh2T = Wh[:, D:].T
    dtaT = dep_type_adj.transpose(0, 1, 3, 2)
    dep_i = deprel_adj.astype(jnp.int32)

    grid_spec = pltpu.PrefetchScalarGridSpec(
        num_scalar_prefetch=1,
        grid=(B, 4),
        in_specs=[
            pl.BlockSpec((1, N + 1, D), lambda b, k, a: (b, 0, 0)),
            pl.BlockSpec(memory_space=pl.ANY),
            pl.BlockSpec((1, N, DE, 128),
                         lambda b, k, a: (b, 0, 0, (a[b] + k) // 128)),
            pl.BlockSpec((1, N, 128),
                         lambda b, k, a: (b, 0, (a[b] + k) // 128)),
            pl.BlockSpec(memory_space=pl.ANY),
            pl.BlockSpec((D, D), lambda b, k, a: (0, 0)),
            pl.BlockSpec((1, D), lambda b, k, a: (0, 0)),
            pl.BlockSpec((DE, 1), lambda b, k, a: (0, 0)),
            pl.BlockSpec((D, 1), lambda b, k, a: (0, 0)),
            pl.BlockSpec((D, 1), lambda b, k, a: (0, 0)),
            pl.BlockSpec((DE, D), lambda b, k, a: (0, 0)),
            pl.BlockSpec((D, D), lambda b, k, a: (0, 0)),
            pl.BlockSpec((D, D), lambda b, k, a: (0, 0)),
            pl.BlockSpec((D, D), lambda b, k, a: (0, 0)),
        ],
        out_specs=pl.BlockSpec((1, N + 1, D), lambda b, k, a: (b, 0, 0)),
        scratch_shapes=[
            pltpu.VMEM((N, D), jnp.float32),
            pltpu.VMEM((N, 1), jnp.float32),
            pltpu.VMEM((N, D), jnp.float32),
            pltpu.VMEM((4, D), jnp.float32),
            pltpu.VMEM((4, 1), jnp.float32),
            pltpu.VMEM((4, D), jnp.float32),
            pltpu.VMEM((DE, N), jnp.float32),
            pltpu.VMEM((16, N), jnp.int32),
            pltpu.SemaphoreType.DMA((2,)),
        ],
    )
    out = pl.pallas_call(
        _body,
        grid_spec=grid_spec,
        out_shape=jax.ShapeDtypeStruct((B, N + 1, D), jnp.float32),
    )(asp_i, features, dtaT, dtaT, dep_i, dep_i,
      wzT, bz2, wae, waf, war, wmeT, wmfT, wh1T, wh2T)
    return out
